# baseline (speedup 1.0000x reference)
import functools

import jax
import jax.numpy as jnp
from jax import lax
from jax.experimental import pallas as pl
from jax.experimental.pallas import tpu as pltpu
from jax.experimental.pallas import tpu_sc as plsc

_NUM_WORDS = 1000000
_D = 64
_B = 4096
_L = 50
_TOTAL = _B * _L
_NC = 2
_NS = 16
_NW = _NC * _NS
_PER_W = _TOTAL // _NW
_STEP = 128
_CSTEPS = 5
_CHUNK = _STEP * _CSTEPS
_NCHUNK = _PER_W // _CHUNK


@functools.partial(
    pl.kernel,
    mesh=plsc.VectorSubcoreMesh(core_axis_name="c", subcore_axis_name="s"),
    out_type=jax.ShapeDtypeStruct((_TOTAL, _D), jnp.float32),
    scratch_types=[
        pltpu.VMEM((_PER_W,), jnp.int32),
        pltpu.VMEM((_CHUNK, _D), jnp.float32),
        pltpu.SemaphoreType.DMA,
    ],
    compiler_params=pltpu.CompilerParams(use_tc_tiling_on_sc=False),
)
def _emb_gather(idx_hbm, table_hbm, out_hbm, idx_v, buf, sem):
    wid = lax.axis_index("s") * _NC + lax.axis_index("c")
    base = wid * _PER_W
    pltpu.sync_copy(idx_hbm.at[pl.ds(base, _PER_W)], idx_v)

    def chunk_body(g, carry):
        off = g * _CHUNK
        cps = []
        for j in range(_CSTEPS):
            cps.append(
                pltpu.async_copy(
                    table_hbm.at[idx_v.at[pl.ds(off + j * _STEP, _STEP)]],
                    buf.at[pl.ds(j * _STEP, _STEP)],
                    sem,
                )
            )
        for cp in cps:
            cp.wait()
        pltpu.sync_copy(buf, out_hbm.at[pl.ds(base + off, _CHUNK)])
        return carry

    lax.fori_loop(0, _NCHUNK, chunk_body, 0)


def kernel(word_ids, word_emb):
    idx = word_ids.reshape(_TOTAL)
    out = _emb_gather(idx, word_emb)
    return out.reshape(_B, _L, _D)


# speedup vs baseline: 1.0083x; 1.0083x over previous
import functools

import jax
import jax.numpy as jnp
from jax import lax
from jax.experimental import pallas as pl
from jax.experimental.pallas import tpu as pltpu
from jax.experimental.pallas import tpu_sc as plsc

_NUM_WORDS = 1000000
_D = 64
_B = 4096
_L = 50
_TOTAL = _B * _L
_NC = 2
_NS = 16
_NW = _NC * _NS
_PER_W = _TOTAL // _NW
_STEP = 128
_NSTREAM = _PER_W // _STEP
_RING = 10
_NOUTER = _NSTREAM // _RING


@functools.partial(
    pl.kernel,
    mesh=plsc.VectorSubcoreMesh(core_axis_name="c", subcore_axis_name="s"),
    out_type=jax.ShapeDtypeStruct((_TOTAL, _D), jnp.float32),
    scratch_types=[
        pltpu.VMEM((_PER_W,), jnp.int32),
        pltpu.VMEM((_RING, _STEP, _D), jnp.float32),
        [pltpu.SemaphoreType.DMA] * _RING,
        [pltpu.SemaphoreType.DMA] * _RING,
    ],
    compiler_params=pltpu.CompilerParams(use_tc_tiling_on_sc=False),
)
def _emb_gather(idx_hbm, table_hbm, out_hbm, idx_v, bufs, gsems, wsems):
    wid = lax.axis_index("s") * _NC + lax.axis_index("c")
    base = wid * _PER_W
    pltpu.sync_copy(idx_hbm.at[pl.ds(base, _PER_W)], idx_v)

    def fire_gather(s, i):
        pltpu.async_copy(
            table_hbm.at[idx_v.at[pl.ds(s * _STEP, _STEP)]],
            bufs.at[i],
            gsems[i],
        )

    for i in range(_RING):
        fire_gather(i, i)

    def outer_body(k, carry):
        for i in range(_RING):
            s = k * _RING + i
            pltpu.make_async_copy(
                table_hbm.at[idx_v.at[pl.ds(s * _STEP, _STEP)]],
                bufs.at[i],
                gsems[i],
            ).wait()
            wcp = pltpu.async_copy(
                bufs.at[i],
                out_hbm.at[pl.ds(base + s * _STEP, _STEP)],
                wsems[i],
            )

            @pl.when(k < _NOUTER - 1)
            def _():
                wcp.wait()
                fire_gather(s + _RING, i)

        return carry

    lax.fori_loop(0, _NOUTER, outer_body, 0)

    for i in range(_RING):
        s = (_NOUTER - 1) * _RING + i
        pltpu.make_async_copy(
            bufs.at[i],
            out_hbm.at[pl.ds(base + s * _STEP, _STEP)],
            wsems[i],
        ).wait()


def kernel(word_ids, word_emb):
    idx = word_ids.reshape(_TOTAL)
    out = _emb_gather(idx, word_emb)
    return out.reshape(_B, _L, _D)
